# baseline (speedup 1.0000x reference)
import functools

import jax
import jax.numpy as jnp
from jax import lax
from jax.experimental import pallas as pl
from jax.experimental.pallas import tpu as pltpu
from jax.experimental.pallas import tpu_sc as plsc

N = 10000
E = 320000
D = 128

NC = 2
NS = 16
L = 16
NT = NC * NS
FPT = D // NT

NP = 10240
EC = 2048
EPAD = 323584
NCHE = EPAD // EC
EPT = EPAD // NT

_vec_mesh = plsc.VectorSubcoreMesh(core_axis_name="c", subcore_axis_name="s")


def _deg_body(dst_hbm, w_hbm, degp_hbm, dst_v, w_v, deg_v):
    c = lax.axis_index("c")
    s = lax.axis_index("s")
    tid = c * NS + s

    zero = jnp.zeros((L,), jnp.float32)

    @pl.loop(0, NP // L, unroll=8)
    def _(i):
        deg_v[pl.ds(i * L, L)] = zero

    pltpu.sync_copy(dst_hbm.at[pl.ds(tid * EPT, EPT)], dst_v)
    pltpu.sync_copy(w_hbm.at[pl.ds(tid * EPT, EPT)], w_v)

    @pl.loop(0, EPT // L, unroll=4)
    def _(i):
        sl = pl.ds(i * L, L)
        plsc.addupdate_scatter(deg_v, [dst_v[sl]], w_v[sl])

    pltpu.sync_copy(deg_v, degp_hbm.at[tid])


@functools.partial(
    pl.kernel,
    out_type=jax.ShapeDtypeStruct((NT, NP), jnp.float32),
    mesh=_vec_mesh,
    compiler_params=pltpu.CompilerParams(needs_layout_passes=False),
    scratch_types=[
        pltpu.VMEM((EPT,), jnp.int32),
        pltpu.VMEM((EPT,), jnp.float32),
        pltpu.VMEM((NP,), jnp.float32),
    ],
)
def _deg_kernel(dst_hbm, w_hbm, degp_hbm, dst_v, w_v, deg_v):
    _deg_body(dst_hbm, w_hbm, degp_hbm, dst_v, w_v, deg_v)


def _prop_body(gt_hbm, src_hbm, dst_hbm, w_hbm, acct_hbm,
               gcs, acs, srcb, dstb, wb, esems, csem):
    c = lax.axis_index("c")
    s = lax.axis_index("s")
    tid = c * NS + s
    frow = tid * FPT

    for f in range(FPT):
        pltpu.async_copy(gt_hbm.at[frow + f], gcs[f], csem)

    zero = jnp.zeros((L,), jnp.float32)
    for f in range(FPT):
        @pl.loop(0, NP // L, unroll=8)
        def _(i):
            acs[f][pl.ds(i * L, L)] = zero

    for f in range(FPT):
        pltpu.make_async_copy(gt_hbm.at[frow + f], gcs[f], csem).wait()

    def issue_edges(k, b):
        sl = pl.ds(k * EC, EC)
        pltpu.async_copy(src_hbm.at[sl], srcb[b], esems[b])
        pltpu.async_copy(dst_hbm.at[sl], dstb[b], esems[b])
        pltpu.async_copy(w_hbm.at[sl], wb[b], esems[b])

    def wait_edges(k, b):
        sl = pl.ds(k * EC, EC)
        pltpu.make_async_copy(src_hbm.at[sl], srcb[b], esems[b]).wait()
        pltpu.make_async_copy(dst_hbm.at[sl], dstb[b], esems[b]).wait()
        pltpu.make_async_copy(w_hbm.at[sl], wb[b], esems[b]).wait()

    issue_edges(0, 0)

    @pl.loop(0, NCHE, step=2)
    def _(g):
        for b in range(2):
            k = g + b

            @pl.when(k + 1 < NCHE)
            def _():
                issue_edges(k + 1, 1 - b)

            wait_edges(k, b)

            @plsc.parallel_loop(0, EC // L, unroll=4)
            def _(i):
                sl = pl.ds(i * L, L)
                src16 = srcb[b][sl]
                dst16 = dstb[b][sl]
                w16 = wb[b][sl]
                for f in range(FPT):
                    v = plsc.load_gather(gcs[f], [src16]) * w16
                    plsc.addupdate_scatter(acs[f], [dst16], v)

    for f in range(FPT):
        pltpu.sync_copy(acs[f], acct_hbm.at[frow + f])


@functools.partial(
    pl.kernel,
    out_type=jax.ShapeDtypeStruct((D, NP), jnp.float32),
    mesh=_vec_mesh,
    compiler_params=pltpu.CompilerParams(needs_layout_passes=False),
    scratch_types=[
        [pltpu.VMEM((NP,), jnp.float32)] * FPT,
        [pltpu.VMEM((NP,), jnp.float32)] * FPT,
        [pltpu.VMEM((EC,), jnp.int32)] * 2,
        [pltpu.VMEM((EC,), jnp.int32)] * 2,
        [pltpu.VMEM((EC,), jnp.float32)] * 2,
        [pltpu.SemaphoreType.DMA] * 2,
        pltpu.SemaphoreType.DMA,
    ],
)
def _prop_kernel(gt_hbm, src_hbm, dst_hbm, w_hbm, acct_hbm,
                 gcs, acs, srcb, dstb, wb, esems, csem):
    _prop_body(gt_hbm, src_hbm, dst_hbm, w_hbm, acct_hbm,
               gcs, acs, srcb, dstb, wb, esems, csem)


RB = 1024


def _dinv_block(degp):
    deg = jnp.sum(degp, axis=0) + 1.0
    return jnp.where(deg > 0, lax.rsqrt(jnp.maximum(deg, 1e-12)), 0.0)


def _mmT(w, xT_or_x, dims):
    return lax.dot_general(w, xT_or_x, (dims, ((), ())),
                           preferred_element_type=jnp.float32,
                           precision=lax.Precision.HIGHEST)


def _tc_g1_body(degp_ref, x_ref, w1_ref, g1t_ref):
    dinv = _dinv_block(degp_ref[...])
    gt = _mmT(w1_ref[...], x_ref[...], ((0,), (1,)))
    g1t_ref[...] = gt * dinv[None, :]


def _tc_g2_body(degp_ref, at_ref, g1t_ref, b1_ref, w2_ref, g2t_ref):
    dinv = _dinv_block(degp_ref[...])
    acc = at_ref[...] + g1t_ref[...]
    x2t = jnp.maximum(acc * dinv[None, :] + b1_ref[...], 0.0)
    g2t_ref[...] = _mmT(w2_ref[...], x2t, ((0,), (0,))) * dinv[None, :]


def _tc_out_body(degp_ref, at_ref, g2t_ref, b2_ref, outt_ref):
    dinv = _dinv_block(degp_ref[...])
    acc = at_ref[...] + g2t_ref[...]
    outt_ref[...] = acc * dinv[None, :] + b2_ref[...]


_degp_spec = pl.BlockSpec((NT, RB), lambda i: (0, i))
_xrows_spec = pl.BlockSpec((RB, D), lambda i: (i, 0))
_t_spec = pl.BlockSpec((D, RB), lambda i: (0, i))
_mat_spec = pl.BlockSpec((D, D), lambda i: (0, 0))
_bcol_spec = pl.BlockSpec((D, 1), lambda i: (0, 0))
_grid = (NP // RB,)

_t_out = jax.ShapeDtypeStruct((D, NP), jnp.float32)

_tc_g1 = pl.pallas_call(
    _tc_g1_body,
    grid=_grid,
    in_specs=[_degp_spec, _xrows_spec, _mat_spec],
    out_specs=_t_spec,
    out_shape=_t_out,
)

_tc_g2 = pl.pallas_call(
    _tc_g2_body,
    grid=_grid,
    in_specs=[_degp_spec, _t_spec, _t_spec, _bcol_spec, _mat_spec],
    out_specs=_t_spec,
    out_shape=_t_out,
)

_tc_out = pl.pallas_call(
    _tc_out_body,
    grid=_grid,
    in_specs=[_degp_spec, _t_spec, _t_spec, _bcol_spec],
    out_specs=_t_spec,
    out_shape=_t_out,
)


def kernel(x, edge_index, edge_attr, W1, b1, W2, b2):
    src = edge_index[0]
    dst = edge_index[1]
    pad = EPAD - E
    pad_idx = jnp.full((pad,), NP - 1, jnp.int32)
    src_f = jnp.concatenate([src, pad_idx])
    dst_f = jnp.concatenate([dst, pad_idx])
    w_f = jnp.concatenate([edge_attr, jnp.zeros((pad,), jnp.float32)])
    x_p = jnp.pad(x, ((0, NP - N), (0, 0)))

    degp = _deg_kernel(dst_f, w_f)
    g1t = _tc_g1(degp, x_p, W1)
    a1t = _prop_kernel(g1t, src_f, dst_f, w_f)
    g2t = _tc_g2(degp, a1t, g1t, b1.reshape(D, 1), W2)
    a2t = _prop_kernel(g2t, src_f, dst_f, w_f)
    outt = _tc_out(degp, a2t, g2t, b2.reshape(D, 1))
    return outt.T[:N]


# speedup vs baseline: 1.0382x; 1.0382x over previous
import functools

import jax
import jax.numpy as jnp
from jax import lax
from jax.experimental import pallas as pl
from jax.experimental.pallas import tpu as pltpu
from jax.experimental.pallas import tpu_sc as plsc

N = 10000
E = 320000
D = 128

NC = 2
NS = 16
L = 16
NT = NC * NS
FPT = D // NT

NP = 10240
EC = 2048
EPAD = 323584
NCHE = EPAD // EC
EPT = EPAD // NT

_vec_mesh = plsc.VectorSubcoreMesh(core_axis_name="c", subcore_axis_name="s")


def _deg_body(dst_hbm, w_hbm, degp_hbm, dst_v, w_v, deg_v):
    c = lax.axis_index("c")
    s = lax.axis_index("s")
    tid = c * NS + s

    zero = jnp.zeros((L,), jnp.float32)

    @pl.loop(0, NP // L, unroll=8)
    def _(i):
        deg_v[pl.ds(i * L, L)] = zero

    pltpu.sync_copy(dst_hbm.at[pl.ds(tid * EPT, EPT)], dst_v)
    pltpu.sync_copy(w_hbm.at[pl.ds(tid * EPT, EPT)], w_v)

    @pl.loop(0, EPT // L, unroll=4)
    def _(i):
        sl = pl.ds(i * L, L)
        plsc.addupdate_scatter(deg_v, [dst_v[sl]], w_v[sl])

    pltpu.sync_copy(deg_v, degp_hbm.at[tid])


@functools.partial(
    pl.kernel,
    out_type=jax.ShapeDtypeStruct((NT, NP), jnp.float32),
    mesh=_vec_mesh,
    compiler_params=pltpu.CompilerParams(needs_layout_passes=False),
    scratch_types=[
        pltpu.VMEM((EPT,), jnp.int32),
        pltpu.VMEM((EPT,), jnp.float32),
        pltpu.VMEM((NP,), jnp.float32),
    ],
)
def _deg_kernel(dst_hbm, w_hbm, degp_hbm, dst_v, w_v, deg_v):
    _deg_body(dst_hbm, w_hbm, degp_hbm, dst_v, w_v, deg_v)


def _prop_body(gt_hbm, sd_hbm, w_hbm, acct_hbm,
               gcs, acs, sdb, wb, esems, csem):
    c = lax.axis_index("c")
    s = lax.axis_index("s")
    tid = c * NS + s
    frow = tid * FPT

    for f in range(FPT):
        pltpu.async_copy(gt_hbm.at[frow + f], gcs[f], csem)

    zero = jnp.zeros((L,), jnp.float32)
    for f in range(FPT):
        @pl.loop(0, NP // L, unroll=8)
        def _(i):
            acs[f][pl.ds(i * L, L)] = zero

    for f in range(FPT):
        pltpu.make_async_copy(gt_hbm.at[frow + f], gcs[f], csem).wait()

    def issue_edges(k, b):
        sl = pl.ds(k * EC, EC)
        pltpu.async_copy(sd_hbm.at[sl], sdb[b], esems[b])
        pltpu.async_copy(w_hbm.at[sl], wb[b], esems[b])

    def wait_edges(k, b):
        sl = pl.ds(k * EC, EC)
        pltpu.make_async_copy(sd_hbm.at[sl], sdb[b], esems[b]).wait()
        pltpu.make_async_copy(w_hbm.at[sl], wb[b], esems[b]).wait()

    issue_edges(0, 0)

    @pl.loop(0, NCHE, step=2)
    def _(g):
        for b in range(2):
            k = g + b
            wait_edges(k, b)

            @pl.when(k + 1 < NCHE)
            def _():
                issue_edges(k + 1, 1 - b)

            @plsc.parallel_loop(0, EC // L, unroll=4)
            def _(i):
                sl = pl.ds(i * L, L)
                sd16 = sdb[b][sl]
                src16 = sd16 & 0x3FFF
                dst16 = lax.shift_right_logical(sd16, 14)
                w16 = wb[b][sl]
                for f in range(FPT):
                    v = plsc.load_gather(gcs[f], [src16]) * w16
                    plsc.addupdate_scatter(acs[f], [dst16], v)

    for f in range(FPT):
        pltpu.sync_copy(acs[f], acct_hbm.at[frow + f])


@functools.partial(
    pl.kernel,
    out_type=jax.ShapeDtypeStruct((D, NP), jnp.float32),
    mesh=_vec_mesh,
    compiler_params=pltpu.CompilerParams(needs_layout_passes=False),
    scratch_types=[
        [pltpu.VMEM((NP,), jnp.float32)] * FPT,
        [pltpu.VMEM((NP,), jnp.float32)] * FPT,
        [pltpu.VMEM((EC,), jnp.int32)] * 2,
        [pltpu.VMEM((EC,), jnp.float32)] * 2,
        [pltpu.SemaphoreType.DMA] * 2,
        pltpu.SemaphoreType.DMA,
    ],
)
def _prop_kernel(gt_hbm, sd_hbm, w_hbm, acct_hbm,
                 gcs, acs, sdb, wb, esems, csem):
    _prop_body(gt_hbm, sd_hbm, w_hbm, acct_hbm,
               gcs, acs, sdb, wb, esems, csem)


RB = 1024


def _dinv_block(degp):
    deg = jnp.sum(degp, axis=0) + 1.0
    return jnp.where(deg > 0, lax.rsqrt(jnp.maximum(deg, 1e-12)), 0.0)


def _mmT(w, xT_or_x, dims):
    return lax.dot_general(w, xT_or_x, (dims, ((), ())),
                           preferred_element_type=jnp.float32,
                           precision=lax.Precision.HIGHEST)


def _tc_g1_body(degp_ref, x_ref, w1_ref, g1t_ref):
    dinv = _dinv_block(degp_ref[...])
    gt = _mmT(w1_ref[...], x_ref[...], ((0,), (1,)))
    g1t_ref[...] = gt * dinv[None, :]


def _tc_g2_body(degp_ref, at_ref, g1t_ref, b1_ref, w2_ref, g2t_ref):
    dinv = _dinv_block(degp_ref[...])
    acc = at_ref[...] + g1t_ref[...]
    x2t = jnp.maximum(acc * dinv[None, :] + b1_ref[...], 0.0)
    g2t_ref[...] = _mmT(w2_ref[...], x2t, ((0,), (0,))) * dinv[None, :]


def _tc_out_body(degp_ref, at_ref, g2t_ref, b2_ref, outt_ref):
    dinv = _dinv_block(degp_ref[...])
    acc = at_ref[...] + g2t_ref[...]
    outt_ref[...] = acc * dinv[None, :] + b2_ref[...]


_degp_spec = pl.BlockSpec((NT, RB), lambda i: (0, i))
_xrows_spec = pl.BlockSpec((RB, D), lambda i: (i, 0))
_t_spec = pl.BlockSpec((D, RB), lambda i: (0, i))
_mat_spec = pl.BlockSpec((D, D), lambda i: (0, 0))
_bcol_spec = pl.BlockSpec((D, 1), lambda i: (0, 0))
_grid = (NP // RB,)

_t_out = jax.ShapeDtypeStruct((D, NP), jnp.float32)

_tc_g1 = pl.pallas_call(
    _tc_g1_body,
    grid=_grid,
    in_specs=[_degp_spec, _xrows_spec, _mat_spec],
    out_specs=_t_spec,
    out_shape=_t_out,
)

_tc_g2 = pl.pallas_call(
    _tc_g2_body,
    grid=_grid,
    in_specs=[_degp_spec, _t_spec, _t_spec, _bcol_spec, _mat_spec],
    out_specs=_t_spec,
    out_shape=_t_out,
)

_tc_out = pl.pallas_call(
    _tc_out_body,
    grid=_grid,
    in_specs=[_degp_spec, _t_spec, _t_spec, _bcol_spec],
    out_specs=_t_spec,
    out_shape=_t_out,
)


def kernel(x, edge_index, edge_attr, W1, b1, W2, b2):
    src = edge_index[0]
    dst = edge_index[1]
    pad = EPAD - E
    pad_idx = jnp.full((pad,), NP - 1, jnp.int32)
    src_f = jnp.concatenate([src, pad_idx])
    dst_f = jnp.concatenate([dst, pad_idx])
    w_f = jnp.concatenate([edge_attr, jnp.zeros((pad,), jnp.float32)])
    x_p = jnp.pad(x, ((0, NP - N), (0, 0)))
    sd_f = src_f | (dst_f << 14)

    degp = _deg_kernel(dst_f, w_f)
    g1t = _tc_g1(degp, x_p, W1)
    a1t = _prop_kernel(g1t, sd_f, w_f)
    g2t = _tc_g2(degp, a1t, g1t, b1.reshape(D, 1), W2)
    a2t = _prop_kernel(g2t, sd_f, w_f)
    outt = _tc_out(degp, a2t, g2t, b2.reshape(D, 1))
    return outt.T[:N]
